# baseline (speedup 1.0000x reference)
import functools
import jax
import jax.numpy as jnp
from jax import lax
from jax.experimental import pallas as pl
from jax.experimental.pallas import tpu as pltpu

D = 256
NB = 2000


def _rowspec(cols):
    return pl.BlockSpec((NB, cols), lambda i: (i, 0))


def _fullspec(shape):
    return pl.BlockSpec(shape, lambda i: tuple(0 for _ in shape))


def _normalize_rows(x):
    n = jnp.sqrt(jnp.sum(x * x, axis=1, keepdims=True))
    return x / jnp.maximum(n, 1e-12)


def _norm_body(x_ref, o_ref):
    o_ref[...] = _normalize_rows(x_ref[...])


def _tc_norm(x):
    return pl.pallas_call(
        _norm_body,
        grid=(x.shape[0] // NB,),
        in_specs=[_rowspec(D)],
        out_specs=_rowspec(D),
        out_shape=jax.ShapeDtypeStruct(x.shape, x.dtype),
    )(x)


def _relstep_body(relsum_ref, relcnt_ref, r0_ref, relp_ref,
                  wir_ref, wic_ref, whh_ref, bih_ref, bhh_ref, o_ref):
    cur = relsum_ref[...] / jnp.maximum(relcnt_ref[...], 1.0)
    gi = (jnp.dot(r0_ref[...], wir_ref[...], preferred_element_type=jnp.float32)
          + jnp.dot(cur, wic_ref[...], preferred_element_type=jnp.float32)
          + bih_ref[...])
    gh = (jnp.dot(relp_ref[...], whh_ref[...], preferred_element_type=jnp.float32)
          + bhh_ref[...])
    i_r, i_z, i_n = gi[:, :D], gi[:, D:2 * D], gi[:, 2 * D:]
    h_r, h_z, h_n = gh[:, :D], gh[:, D:2 * D], gh[:, 2 * D:]
    r = jax.nn.sigmoid(i_r + h_r)
    z = jax.nn.sigmoid(i_z + h_z)
    n = jnp.tanh(i_n + r * h_n)
    o_ref[...] = _normalize_rows((1.0 - z) * n + z * relp_ref[...])


def _tc_relstep(relsum, relcnt, r0, relp, wir, wic, whh, bih, bhh):
    m = relsum.shape[0]
    return pl.pallas_call(
        _relstep_body,
        grid=(m // NB,),
        in_specs=[_rowspec(D), pl.BlockSpec((NB, 1), lambda i: (i, 0)),
                  _rowspec(D), _rowspec(D),
                  _fullspec((D, 3 * D)), _fullspec((D, 3 * D)), _fullspec((D, 3 * D)),
                  _fullspec((1, 3 * D)), _fullspec((1, 3 * D))],
        out_specs=_rowspec(D),
        out_shape=jax.ShapeDtypeStruct((m, D), jnp.float32),
    )(relsum, relcnt, r0, relp, wir, wic, whh, bih, bhh)


def _prep_body(h_ref, rel_ref, wn_ref, wl_ref, hw_ref, relw_ref, hlw_ref):
    hw_ref[...] = jnp.dot(h_ref[...], wn_ref[...], preferred_element_type=jnp.float32)
    relw_ref[...] = jnp.dot(rel_ref[...], wn_ref[...], preferred_element_type=jnp.float32)
    hlw_ref[...] = jnp.dot(h_ref[...], wl_ref[...], preferred_element_type=jnp.float32)


def _tc_prep(h, rel, wn, wl):
    m = h.shape[0]
    sd = jax.ShapeDtypeStruct((m, D), jnp.float32)
    return pl.pallas_call(
        _prep_body,
        grid=(m // NB,),
        in_specs=[_rowspec(D), _rowspec(D), _fullspec((D, D)), _fullspec((D, D))],
        out_specs=(_rowspec(D), _rowspec(D), _rowspec(D)),
        out_shape=(sd, sd, sd),
    )(h, rel, wn, wl)


def _prep2_body(acc_ref, deg_ref, hlw_ref, rel_ref, wn_ref, wl_ref,
                hw_ref, relw_ref, hlw2_ref):
    h2 = jax.nn.relu(acc_ref[...] / jnp.maximum(deg_ref[...], 1.0) + hlw_ref[...])
    hw_ref[...] = jnp.dot(h2, wn_ref[...], preferred_element_type=jnp.float32)
    relw_ref[...] = jnp.dot(rel_ref[...], wn_ref[...], preferred_element_type=jnp.float32)
    hlw2_ref[...] = jnp.dot(h2, wl_ref[...], preferred_element_type=jnp.float32)


def _tc_prep2(acc, deg, hlw, rel, wn, wl):
    m = acc.shape[0]
    sd = jax.ShapeDtypeStruct((m, D), jnp.float32)
    return pl.pallas_call(
        _prep2_body,
        grid=(m // NB,),
        in_specs=[_rowspec(D), pl.BlockSpec((NB, 1), lambda i: (i, 0)), _rowspec(D),
                  _rowspec(D), _fullspec((D, D)), _fullspec((D, D))],
        out_specs=(_rowspec(D), _rowspec(D), _rowspec(D)),
        out_shape=(sd, sd, sd),
    )(acc, deg, hlw, rel, wn, wl)


def _final_body(acc_ref, deg_ref, hlw_ref, ent_ref, gw_ref, gb_ref, o_ref):
    cur = _normalize_rows(
        jax.nn.relu(acc_ref[...] / jnp.maximum(deg_ref[...], 1.0) + hlw_ref[...]))
    gate = jax.nn.sigmoid(
        jnp.dot(ent_ref[...], gw_ref[...], preferred_element_type=jnp.float32)
        + gb_ref[...])
    o_ref[...] = gate * cur + (1.0 - gate) * ent_ref[...]


def _tc_final(acc, deg, hlw, ent, gw, gb):
    m = acc.shape[0]
    return pl.pallas_call(
        _final_body,
        grid=(m // NB,),
        in_specs=[_rowspec(D), pl.BlockSpec((NB, 1), lambda i: (i, 0)), _rowspec(D),
                  _rowspec(D), _fullspec((D, D)), _fullspec((1, D))],
        out_specs=_rowspec(D),
        out_shape=jax.ShapeDtypeStruct((m, D), jnp.float32),
    )(acc, deg, hlw, ent, gw, gb)


def _avg_rela_sums(e_all, r_all, ent, num_ent, num_rel):
    key = e_all * num_rel + r_all
    pid = jnp.arange(e_all.shape[0], dtype=jnp.int32)
    T = jnp.zeros((num_ent * num_rel,), jnp.int32)
    T = T.at[key].set(pid, mode="drop")
    win = (T[key] == pid).astype(jnp.float32)
    relsum = jax.ops.segment_sum(ent[e_all] * win[:, None], r_all, num_segments=num_rel)
    relcnt = jax.ops.segment_sum(win, r_all, num_segments=num_rel)
    return relsum, relcnt


def _edge_agg(hw, relw, src, rtype, dst, num_ent):
    return jax.ops.segment_sum(hw[src] + relw[rtype], dst, num_segments=num_ent)


def kernel(edges, static_entity_embed, static_relation_embed, gate_weight, gate_bias,
           gru_w_ih, gru_w_hh, gru_b_ih, gru_b_hh, rgcn_w_neigh, rgcn_w_loop):
    num_ent = static_entity_embed.shape[0]
    num_rel = static_relation_embed.shape[0]
    num_layer = rgcn_w_neigh.shape[0]
    R0 = static_relation_embed
    wir = gru_w_ih[:, :D].T
    wic = gru_w_ih[:, D:].T
    whh = gru_w_hh.T
    bih = gru_b_ih.reshape(1, 3 * D)
    bhh = gru_b_hh.reshape(1, 3 * D)
    gb = gate_bias.reshape(1, D)

    ent = _tc_norm(static_entity_embed)
    rel = R0
    for t in range(edges.shape[0]):
        edge = edges[t]
        src, rtype, dst = edge[:, 0], edge[:, 1], edge[:, 2]
        e_all = jnp.concatenate([src, dst])
        r_all = jnp.concatenate([rtype, rtype])
        relsum, relcnt = _avg_rela_sums(e_all, r_all, ent, num_ent, num_rel)
        rel = _tc_relstep(relsum, relcnt.reshape(num_rel, 1), R0, rel,
                          wir, wic, whh, bih, bhh)
        deg = jax.ops.segment_sum(jnp.ones((src.shape[0],), jnp.float32), dst,
                                  num_segments=num_ent).reshape(num_ent, 1)
        hw, relw, hlw = _tc_prep(ent, rel, rgcn_w_neigh[0], rgcn_w_loop[0])
        for l in range(1, num_layer):
            acc = _edge_agg(hw, relw, src, rtype, dst, num_ent)
            hw, relw, hlw = _tc_prep2(acc, deg, hlw, rel,
                                      rgcn_w_neigh[l], rgcn_w_loop[l])
        acc = _edge_agg(hw, relw, src, rtype, dst, num_ent)
        ent = _tc_final(acc, deg, hlw, ent, gate_weight, gb)
    ent = _tc_norm(ent)
    return ent, rel


# speedup vs baseline: 1.1169x; 1.1169x over previous
import functools
import jax
import jax.numpy as jnp
from jax import lax
from jax.experimental import pallas as pl
from jax.experimental.pallas import tpu as pltpu
from jax.experimental.pallas import tpu_sc as plsc

D = 256
NB = 2000
NC, NS, L = 2, 16, 16

E_EDGES = 160000
NPAIR = 2 * E_EDGES
NUM_REL2 = 10000
ROWS = 10240
TRASH = 10000
BP = 80
EPAD = 161280
TKEYS = 100_000_000
SH = 2
BLK = 512


def _rowspec(cols):
    return pl.BlockSpec((NB, cols), lambda i: (i, 0))


def _accspec(cols):
    return pl.BlockSpec((SH, NB, cols), lambda i: (0, i, 0))


def _fullspec(shape):
    return pl.BlockSpec(shape, lambda i: tuple(0 for _ in shape))


def _normalize_rows(x):
    n = jnp.sqrt(jnp.sum(x * x, axis=1, keepdims=True))
    return x / jnp.maximum(n, 1e-12)


def _shsum(ref):
    x = ref[0]
    for k in range(1, SH):
        x = x + ref[k]
    return x


def _norm_body(x_ref, o_ref):
    o_ref[...] = _normalize_rows(x_ref[...])


def _tc_norm(x):
    return pl.pallas_call(
        _norm_body,
        grid=(x.shape[0] // NB,),
        in_specs=[_rowspec(D)],
        out_specs=_rowspec(D),
        out_shape=jax.ShapeDtypeStruct(x.shape, x.dtype),
    )(x)


def _relstep_body(rs_ref, ct_ref, r0_ref, relp_ref,
                  wir_ref, wic_ref, whh_ref, bih_ref, bhh_ref, o_ref):
    relsum = _shsum(rs_ref)
    cnt = _shsum(ct_ref)[:, 0:1]
    cur = relsum / jnp.maximum(cnt, 1.0)
    gi = (jnp.dot(r0_ref[...], wir_ref[...], preferred_element_type=jnp.float32)
          + jnp.dot(cur, wic_ref[...], preferred_element_type=jnp.float32)
          + bih_ref[...])
    gh = (jnp.dot(relp_ref[...], whh_ref[...], preferred_element_type=jnp.float32)
          + bhh_ref[...])
    i_r, i_z, i_n = gi[:, :D], gi[:, D:2 * D], gi[:, 2 * D:]
    h_r, h_z, h_n = gh[:, :D], gh[:, D:2 * D], gh[:, 2 * D:]
    r = jax.nn.sigmoid(i_r + h_r)
    z = jax.nn.sigmoid(i_z + h_z)
    n = jnp.tanh(i_n + r * h_n)
    o_ref[...] = _normalize_rows((1.0 - z) * n + z * relp_ref[...])


def _tc_relstep(rs, ct, r0, relp, wir, wic, whh, bih, bhh):
    m = r0.shape[0]
    return pl.pallas_call(
        _relstep_body,
        grid=(m // NB,),
        in_specs=[_accspec(D), _accspec(8),
                  _rowspec(D), _rowspec(D),
                  _fullspec((D, 3 * D)), _fullspec((D, 3 * D)), _fullspec((D, 3 * D)),
                  _fullspec((1, 3 * D)), _fullspec((1, 3 * D))],
        out_specs=_rowspec(D),
        out_shape=jax.ShapeDtypeStruct((m, D), jnp.float32),
    )(rs, ct, r0, relp, wir, wic, whh, bih, bhh)


def _prep_body(h_ref, rel_ref, wn_ref, wl_ref, hw_ref, relw_ref, hlw_ref):
    hw_ref[...] = jnp.dot(h_ref[...], wn_ref[...], preferred_element_type=jnp.float32)
    relw_ref[...] = jnp.dot(rel_ref[...], wn_ref[...], preferred_element_type=jnp.float32)
    hlw_ref[...] = jnp.dot(h_ref[...], wl_ref[...], preferred_element_type=jnp.float32)


def _tc_prep(h, rel, wn, wl):
    m = h.shape[0]
    sd = jax.ShapeDtypeStruct((m, D), jnp.float32)
    return pl.pallas_call(
        _prep_body,
        grid=(m // NB,),
        in_specs=[_rowspec(D), _rowspec(D), _fullspec((D, D)), _fullspec((D, D))],
        out_specs=(_rowspec(D), _rowspec(D), _rowspec(D)),
        out_shape=(sd, sd, sd),
    )(h, rel, wn, wl)


def _prep2_body(a_ref, d_ref, hlw_ref, rel_ref, wn_ref, wl_ref,
                hw_ref, relw_ref, hlw2_ref):
    acc = _shsum(a_ref)
    deg = _shsum(d_ref)[:, 0:1]
    h2 = jax.nn.relu(acc / jnp.maximum(deg, 1.0) + hlw_ref[...])
    hw_ref[...] = jnp.dot(h2, wn_ref[...], preferred_element_type=jnp.float32)
    relw_ref[...] = jnp.dot(rel_ref[...], wn_ref[...], preferred_element_type=jnp.float32)
    hlw2_ref[...] = jnp.dot(h2, wl_ref[...], preferred_element_type=jnp.float32)


def _tc_prep2(a, d, hlw, rel, wn, wl):
    m = hlw.shape[0]
    sd = jax.ShapeDtypeStruct((m, D), jnp.float32)
    return pl.pallas_call(
        _prep2_body,
        grid=(m // NB,),
        in_specs=[_accspec(D), _accspec(8), _rowspec(D),
                  _rowspec(D), _fullspec((D, D)), _fullspec((D, D))],
        out_specs=(_rowspec(D), _rowspec(D), _rowspec(D)),
        out_shape=(sd, sd, sd),
    )(a, d, hlw, rel, wn, wl)


def _final_body(a_ref, d_ref, hlw_ref, ent_ref, gw_ref, gb_ref, o_ref):
    acc = _shsum(a_ref)
    deg = _shsum(d_ref)[:, 0:1]
    cur = _normalize_rows(
        jax.nn.relu(acc / jnp.maximum(deg, 1.0) + hlw_ref[...]))
    gate = jax.nn.sigmoid(
        jnp.dot(ent_ref[...], gw_ref[...], preferred_element_type=jnp.float32)
        + gb_ref[...])
    o_ref[...] = gate * cur + (1.0 - gate) * ent_ref[...]


def _tc_final(a, d, hlw, ent, gw, gb):
    m = hlw.shape[0]
    return pl.pallas_call(
        _final_body,
        grid=(m // NB,),
        in_specs=[_accspec(D), _accspec(8), _rowspec(D),
                  _rowspec(D), _fullspec((D, D)), _fullspec((1, D))],
        out_specs=_rowspec(D),
        out_shape=jax.ShapeDtypeStruct((m, D), jnp.float32),
    )(a, d, hlw, ent, gw, gb)


def _acc_body_factory(with_cnt, use_win):
    def body(*refs):
        if use_win and with_cnt:
            idx_ref, win_ref, msg_ref, acc_ref, cnt_ref = refs
        elif with_cnt:
            idx_ref, msg_ref, acc_ref, cnt_ref = refs
            win_ref = None
        else:
            idx_ref, msg_ref, acc_ref = refs
            win_ref = None

        @pl.when(pl.program_id(0) == 0)
        def _():
            acc_ref[...] = jnp.zeros((SH, ROWS, D), jnp.float32)
            if with_cnt:
                cnt_ref[...] = jnp.zeros((SH, ROWS, 8), jnp.float32)

        def it(m, c):
            for k in range(SH):
                i = m * SH + k
                d = idx_ref[i]
                row = msg_ref[pl.ds(i, 1), :]
                if use_win:
                    w = win_ref[i]
                    row = row * w
                acc_ref[k, pl.ds(d, 1), :] = acc_ref[k, pl.ds(d, 1), :] + row
                if with_cnt:
                    wv = win_ref[i] if use_win else 1.0
                    cnt_ref[k, pl.ds(d, 1), :] = cnt_ref[k, pl.ds(d, 1), :] + wv
            return c

        lax.fori_loop(0, BLK // SH, it, 0)
    return body


def _tc_segsum(idx, msg, win, with_cnt):
    n = msg.shape[0]
    use_win = win is not None
    in_specs = [pl.BlockSpec((BLK,), lambda i: (i,), memory_space=pltpu.SMEM)]
    args = [idx]
    if use_win:
        in_specs.append(pl.BlockSpec((BLK,), lambda i: (i,),
                                     memory_space=pltpu.SMEM))
        args.append(win)
    in_specs.append(pl.BlockSpec((BLK, D), lambda i: (i, 0)))
    args.append(msg)
    acc_sd = jax.ShapeDtypeStruct((SH, ROWS, D), jnp.float32)
    cnt_sd = jax.ShapeDtypeStruct((SH, ROWS, 8), jnp.float32)
    acc_spec = pl.BlockSpec((SH, ROWS, D), lambda i: (0, 0, 0))
    cnt_spec = pl.BlockSpec((SH, ROWS, 8), lambda i: (0, 0, 0))
    if with_cnt:
        return pl.pallas_call(
            _acc_body_factory(True, use_win),
            grid=(n // BLK,),
            in_specs=in_specs,
            out_specs=(acc_spec, cnt_spec),
            out_shape=(acc_sd, cnt_sd),
        )(*args)
    return pl.pallas_call(
        _acc_body_factory(False, use_win),
        grid=(n // BLK,),
        in_specs=in_specs,
        out_specs=acc_spec,
        out_shape=acc_sd,
    )(*args)


def _sc_mesh():
    return plsc.VectorSubcoreMesh(core_axis_name="c", subcore_axis_name="s",
                                  num_cores=NC, num_subcores=NS)


def _wid():
    return lax.axis_index("c") * NS + lax.axis_index("s")


def _iota16():
    return lax.iota(jnp.int32, L)


def _sc_a1_body(e_all, r_all, t_out, keys_out, ebuf, rbuf, keybuf, valbuf, sem):
    wid = _wid()

    def block(b, c):
        off = wid * (NPAIR // (NC * NS)) + b * BP
        pltpu.sync_copy(e_all.at[pl.ds(off, BP)], ebuf)
        pltpu.sync_copy(r_all.at[pl.ds(off, BP)], rbuf)
        for g in range(BP // L):
            sl = pl.ds(g * L, L)
            keybuf[sl] = ebuf[sl] * NUM_REL2 + rbuf[sl]
            valbuf[sl] = off + g * L + _iota16()
        pltpu.sync_copy(keybuf, keys_out.at[pl.ds(off, BP)])
        pltpu.sync_copy(valbuf, t_out.at[keybuf])
        return c

    lax.fori_loop(0, NPAIR // (NC * NS) // BP, block, 0)


def _sc_a1(e_all, r_all):
    f = pl.kernel(
        _sc_a1_body,
        out_type=(jax.ShapeDtypeStruct((TKEYS,), jnp.int32),
                  jax.ShapeDtypeStruct((NPAIR,), jnp.int32)),
        mesh=_sc_mesh(),
        scratch_types=[
            pltpu.VMEM((BP,), jnp.int32), pltpu.VMEM((BP,), jnp.int32),
            pltpu.VMEM((BP,), jnp.int32), pltpu.VMEM((BP,), jnp.int32),
            pltpu.SemaphoreType.DMA,
        ],
    )
    return f(e_all, r_all)


def _sc_a2_body(keys, e_all, t_in, ent, msg_out, win_out,
                keybuf, ebuf, tbuf, winbuf, entrows, sem):
    wid = _wid()

    def block(b, c):
        off = wid * (NPAIR // (NC * NS)) + b * BP
        pltpu.sync_copy(keys.at[pl.ds(off, BP)], keybuf)
        pltpu.sync_copy(e_all.at[pl.ds(off, BP)], ebuf)
        pltpu.async_copy(t_in.at[keybuf], tbuf, sem).wait()
        pltpu.async_copy(ent.at[ebuf], entrows, sem).wait()
        for g in range(BP // L):
            sl = pl.ds(g * L, L)
            pid = off + g * L + _iota16()
            winbuf[sl] = jnp.where(tbuf[sl] == pid, 1.0, 0.0)
        pltpu.sync_copy(entrows, msg_out.at[pl.ds(off, BP)])
        pltpu.sync_copy(winbuf, win_out.at[pl.ds(off, BP)])
        return c

    lax.fori_loop(0, NPAIR // (NC * NS) // BP, block, 0)


def _sc_a2(keys, e_all, t_in, ent):
    f = pl.kernel(
        _sc_a2_body,
        out_type=(jax.ShapeDtypeStruct((NPAIR, D), jnp.float32),
                  jax.ShapeDtypeStruct((NPAIR,), jnp.float32)),
        mesh=_sc_mesh(),
        scratch_types=[
            pltpu.VMEM((BP,), jnp.int32), pltpu.VMEM((BP,), jnp.int32),
            pltpu.VMEM((BP,), jnp.int32), pltpu.VMEM((BP,), jnp.float32),
            pltpu.VMEM((BP, D), jnp.float32),
            pltpu.SemaphoreType.DMA,
        ],
    )
    return f(keys, e_all, t_in, ent)


def _sc_edge_msg_body(srcp, rtp, hw, relw, msg_out,
                      sbuf, tbuf, rows1, rows2, sem):
    wid = _wid()

    def block(b, c):
        off = wid * (EPAD // (NC * NS)) + b * BP
        pltpu.sync_copy(srcp.at[pl.ds(off, BP)], sbuf)
        pltpu.sync_copy(rtp.at[pl.ds(off, BP)], tbuf)
        pltpu.async_copy(hw.at[sbuf], rows1, sem).wait()
        pltpu.async_copy(relw.at[tbuf], rows2, sem).wait()

        def radd(i, c2):
            for g in range(D // L):
                sl = pl.ds(g * L, L)
                rows1[i, sl] = rows1[i, sl] + rows2[i, sl]
            return c2

        lax.fori_loop(0, BP, radd, 0)
        pltpu.sync_copy(rows1, msg_out.at[pl.ds(off, BP)])
        return c

    lax.fori_loop(0, EPAD // (NC * NS) // BP, block, 0)


def _sc_edge_msg(srcp, rtp, hw, relw):
    f = pl.kernel(
        _sc_edge_msg_body,
        out_type=jax.ShapeDtypeStruct((EPAD, D), jnp.float32),
        mesh=_sc_mesh(),
        scratch_types=[
            pltpu.VMEM((BP,), jnp.int32), pltpu.VMEM((BP,), jnp.int32),
            pltpu.VMEM((BP, D), jnp.float32), pltpu.VMEM((BP, D), jnp.float32),
            pltpu.SemaphoreType.DMA,
        ],
    )
    return f(srcp, rtp, hw, relw)


def kernel(edges, static_entity_embed, static_relation_embed, gate_weight, gate_bias,
           gru_w_ih, gru_w_hh, gru_b_ih, gru_b_hh, rgcn_w_neigh, rgcn_w_loop):
    num_layer = rgcn_w_neigh.shape[0]
    R0 = static_relation_embed
    wir = gru_w_ih[:, :D].T
    wic = gru_w_ih[:, D:].T
    whh = gru_w_hh.T
    bih = gru_b_ih.reshape(1, 3 * D)
    bhh = gru_b_hh.reshape(1, 3 * D)
    gb = gate_bias.reshape(1, D)
    padn = EPAD - E_EDGES
    pad0 = jnp.zeros((padn,), jnp.int32)
    padt = jnp.full((padn,), TRASH, jnp.int32)

    ent = _tc_norm(static_entity_embed)
    rel = R0
    for t in range(edges.shape[0]):
        edge = edges[t]
        src, rtype, dst = edge[:, 0], edge[:, 1], edge[:, 2]
        e_all = jnp.concatenate([src, dst])
        r_all = jnp.concatenate([rtype, rtype])
        t_tab, keys = _sc_a1(e_all, r_all)
        amsg, win = _sc_a2(keys, e_all, t_tab, ent)
        rs, ct = _tc_segsum(r_all, amsg, win, True)
        rel = _tc_relstep(rs, ct, R0, rel, wir, wic, whh, bih, bhh)
        srcp = jnp.concatenate([src, pad0])
        rtp = jnp.concatenate([rtype, pad0])
        dstp = jnp.concatenate([dst, padt])
        hw, relw, hlw = _tc_prep(ent, rel, rgcn_w_neigh[0], rgcn_w_loop[0])
        emsg = _sc_edge_msg(srcp, rtp, hw, relw)
        a, d = _tc_segsum(dstp, emsg, None, True)
        for l in range(1, num_layer):
            hw, relw, hlw = _tc_prep2(a, d, hlw, rel,
                                      rgcn_w_neigh[l], rgcn_w_loop[l])
            emsg = _sc_edge_msg(srcp, rtp, hw, relw)
            a = _tc_segsum(dstp, emsg, None, False)
        ent = _tc_final(a, d, hlw, ent, gate_weight, gb)
    ent = _tc_norm(ent)
    return ent, rel
